# baseline (speedup 1.0000x reference)
import functools

import jax
import jax.numpy as jnp
from jax import lax
from jax.experimental import pallas as pl
from jax.experimental.pallas import tpu as pltpu
from jax.experimental.pallas import tpu_sc as plsc

_K = 8
_E = 64
_LANES = 16


@functools.lru_cache(maxsize=1)
def _noise_const(B, L, E):
    n = jax.random.normal(jax.random.key(1), (B, L, E), jnp.float32)
    return n.reshape(B * L, E)


def _logits_body(x_ref, wt_ref, b_ref, noise_ref, noisy_ref):
    z = jnp.dot(x_ref[...], wt_ref[...], preferred_element_type=jnp.float32)
    z = z + b_ref[...]
    logits = z[:, :_E]
    noise_logits = z[:, _E:]
    sp = jnp.maximum(noise_logits, 0.0) + jnp.log1p(jnp.exp(-jnp.abs(noise_logits)))
    noisy_ref[...] = logits + noise_ref[...] * sp


_GATHER_DNUMS = lax.GatherDimensionNumbers(
    offset_dims=(), collapsed_slice_dims=(0,), start_index_map=(0,))


def _rotl(v, idx):
    return lax.gather(v, idx[:, None], _GATHER_DNUMS, slice_sizes=(1,),
                      mode=lax.GatherScatterMode.PROMISE_IN_BOUNDS)


def _transpose16(rows, lane):
    n = _LANES
    cur = rows
    k = 1
    while k < n:
        idx_a = jnp.bitwise_and(lane + k, n - 1)
        idx_b = jnp.bitwise_and(lane + (n - k), n - 1)
        nxt = [None] * n
        for i in range(n):
            p = i ^ k
            if i & k == 0:
                take_self = (lane & k) == 0
                part = _rotl(cur[p], idx_a)
            else:
                take_self = (lane & k) != 0
                part = _rotl(cur[p], idx_b)
            nxt[i] = jnp.where(take_self, cur[i], part)
        cur = nxt
        k *= 2
    return cur


def _route_body(tpw, noisy_flat, outp, idxT, stage_v, vals_v, outp_v, idx_v, sem):
    wid = lax.axis_index("s") * 2 + lax.axis_index("c")
    base = wid * tpw
    cp_in = pltpu.async_copy(
        noisy_flat.at[pl.ds(base * _E, tpw * _E)], stage_v, sem)

    zero16 = jnp.zeros((_LANES,), jnp.float32)

    def zbody(i, c):
        outp_v[pl.ds(pl.multiple_of(i * _LANES, _LANES), _LANES)] = zero16
        return c

    lax.fori_loop(0, tpw * _E // _LANES, zbody, 0)
    cp_in.wait()

    lane = lax.broadcasted_iota(jnp.int32, (_LANES,), 0)
    neg_inf = jnp.full((_LANES,), -jnp.inf, jnp.float32)

    def tbody(g, c):
        t0 = pl.multiple_of(g * _LANES, _LANES)
        for j in range(_E // _LANES):
            rows = [
                stage_v[pl.ds(pl.multiple_of((t0 + i) * _E + j * _LANES, _LANES),
                              _LANES)]
                for i in range(_LANES)
            ]
            cols = _transpose16(rows, lane)
            for r in range(_LANES):
                vals_v[pl.ds(
                    pl.multiple_of((j * _LANES + r) * tpw + t0, _LANES),
                    _LANES)] = cols[r]
        return c

    lax.fori_loop(0, tpw // _LANES, tbody, 0)

    def gbody(g, c):
        col0 = pl.multiple_of(g * _LANES, _LANES)
        tok = g * _LANES + lane
        ms, mis = [], []
        for _ in range(_K):
            def ebody(e, carry):
                m, mi = carry
                v = vals_v[pl.ds(e * tpw + col0, _LANES)]
                better = v > m
                return (jnp.where(better, v, m),
                        jnp.where(better, jnp.full((_LANES,), e, jnp.int32), mi))

            m, mi = lax.fori_loop(
                0, _E, ebody,
                (neg_inf, jnp.zeros((_LANES,), jnp.int32)), unroll=8)
            plsc.store_scatter(vals_v, [mi * tpw + tok], neg_inf)
            ms.append(m)
            mis.append(mi)

        m0 = ms[0]
        ws = [jnp.exp(m - m0) for m in ms]
        denom = ws[0]
        for w in ws[1:]:
            denom = denom + w
        inv = 1.0 / denom
        for k in range(_K):
            plsc.store_scatter(outp_v, [tok * _E + mis[k]], ws[k] * inv)
            idx_v[pl.ds(k * tpw + col0, _LANES)] = mis[k]
        return c

    lax.fori_loop(0, tpw // _LANES, gbody, 0)

    pltpu.sync_copy(outp_v, outp.at[pl.ds(base * _E, tpw * _E)])
    cps = [
        pltpu.async_copy(
            idx_v.at[pl.ds(k * tpw, tpw)],
            idxT.at[k, pl.ds(base, tpw)], sem)
        for k in range(_K)
    ]
    for cp in cps:
        cp.wait()


def kernel(x_BLD, W_route, b_route, W_noise, b_noise):
    B, L, D = x_BLD.shape
    E = W_route.shape[0]
    N = B * L
    T = 1024
    assert N % T == 0 and E == _E

    info = plsc.get_sparse_core_info()
    nw = info.num_cores * info.num_subcores
    tpw = N // nw
    spb = T // tpw

    x = x_BLD.reshape(N, D)
    wt = jnp.concatenate([W_route, W_noise], axis=0).T
    b = jnp.concatenate([b_route, b_noise]).reshape(1, 2 * E)
    noise = _noise_const(B, L, E)

    noisy = pl.pallas_call(
        _logits_body,
        grid=(N // T,),
        in_specs=[
            pl.BlockSpec((T, D), lambda i: (i, 0)),
            pl.BlockSpec((D, 2 * E), lambda i: (0, 0)),
            pl.BlockSpec((1, 2 * E), lambda i: (0, 0)),
            pl.BlockSpec((T, E), lambda i: (i, 0)),
        ],
        out_specs=pl.BlockSpec((T, E), lambda i: (i, 0)),
        out_shape=jax.ShapeDtypeStruct((N, E), jnp.float32),
        compiler_params=pltpu.CompilerParams(
            dimension_semantics=("arbitrary",),
        ),
    )(x, wt, b, noise)

    route = functools.partial(
        pl.kernel,
        out_type=[
            jax.ShapeDtypeStruct((N * E,), jnp.float32),
            jax.ShapeDtypeStruct((_K, N), jnp.int32),
        ],
        scratch_types=[
            pltpu.VMEM((tpw * E,), jnp.float32),
            pltpu.VMEM((E * tpw,), jnp.float32),
            pltpu.VMEM((tpw * E,), jnp.float32),
            pltpu.VMEM((_K * tpw,), jnp.int32),
            pltpu.SemaphoreType.DMA,
        ],
        mesh=plsc.VectorSubcoreMesh(core_axis_name="c", subcore_axis_name="s"),
        compiler_params=pltpu.CompilerParams(needs_layout_passes=False),
    )(functools.partial(_route_body, tpw))

    outp, idxT = route(noisy.reshape(N * E))
    return outp.reshape(B, L, E), idxT.T.reshape(B, L, _K)


# speedup vs baseline: 1.2224x; 1.2224x over previous
import functools

import jax
import jax.numpy as jnp
from jax import lax
from jax.experimental import pallas as pl
from jax.experimental.pallas import tpu as pltpu
from jax.experimental.pallas import tpu_sc as plsc

_K = 8
_E = 64
_LANES = 16


@functools.lru_cache(maxsize=1)
def _noise_const_T(B, L, E):
    n = jax.random.normal(jax.random.key(1), (B, L, E), jnp.float32)
    return n.reshape(B * L, E).T


def _logits_body(x_ref, wt_ref, b_ref, noiseT_ref, noisyT_ref):
    z = jnp.dot(x_ref[...], wt_ref[...], preferred_element_type=jnp.float32)
    z = (z + b_ref[...]).T
    logits = z[:_E, :]
    noise_logits = z[_E:, :]
    sp = jnp.maximum(noise_logits, 0.0) + jnp.log1p(jnp.exp(-jnp.abs(noise_logits)))
    noisyT_ref[...] = logits + noiseT_ref[...] * sp


def _route_body(tpw, noisyT, outp, idxT, vals_v, outp_v, idx_v, sem):
    wid = lax.axis_index("s") * 2 + lax.axis_index("c")
    base = wid * tpw
    cps_in = [
        pltpu.async_copy(
            noisyT.at[e, pl.ds(base, tpw)],
            vals_v.at[pl.ds(e * tpw, tpw)], sem)
        for e in range(_E)
    ]

    zero16 = jnp.zeros((_LANES,), jnp.float32)

    def zbody(i, c):
        outp_v[pl.ds(pl.multiple_of(i * _LANES, _LANES), _LANES)] = zero16
        return c

    lax.fori_loop(0, tpw * _E // _LANES, zbody, 0)
    for cp in cps_in:
        cp.wait()

    lane = lax.broadcasted_iota(jnp.int32, (_LANES,), 0)
    neg_inf = jnp.full((_LANES,), -jnp.inf, jnp.float32)

    def gbody(g, c):
        col0 = pl.multiple_of(g * _LANES, _LANES)
        tok = g * _LANES + lane
        ms, mis = [], []
        for _ in range(_K):
            def ebody(e, carry):
                m, mi = carry
                v = vals_v[pl.ds(e * tpw + col0, _LANES)]
                better = v > m
                return (jnp.where(better, v, m),
                        jnp.where(better, jnp.full((_LANES,), e, jnp.int32), mi))

            m, mi = lax.fori_loop(
                0, _E, ebody,
                (neg_inf, jnp.zeros((_LANES,), jnp.int32)), unroll=8)
            plsc.store_scatter(vals_v, [mi * tpw + tok], neg_inf)
            ms.append(m)
            mis.append(mi)

        m0 = ms[0]
        ws = [jnp.exp(m - m0) for m in ms]
        denom = ws[0]
        for w in ws[1:]:
            denom = denom + w
        inv = 1.0 / denom
        for k in range(_K):
            plsc.store_scatter(outp_v, [tok * _E + mis[k]], ws[k] * inv)
            idx_v[pl.ds(k * tpw + col0, _LANES)] = mis[k]
        return c

    lax.fori_loop(0, tpw // _LANES, gbody, 0)

    pltpu.sync_copy(outp_v, outp.at[pl.ds(base * _E, tpw * _E)])
    cps = [
        pltpu.async_copy(
            idx_v.at[pl.ds(k * tpw, tpw)],
            idxT.at[k, pl.ds(base, tpw)], sem)
        for k in range(_K)
    ]
    for cp in cps:
        cp.wait()


def kernel(x_BLD, W_route, b_route, W_noise, b_noise):
    B, L, D = x_BLD.shape
    E = W_route.shape[0]
    N = B * L
    T = 1024
    assert N % T == 0 and E == _E

    x = x_BLD.reshape(N, D)
    wt = jnp.concatenate([W_route, W_noise], axis=0).T
    b = jnp.concatenate([b_route, b_noise]).reshape(1, 2 * E)
    noiseT = _noise_const_T(B, L, E)

    noisyT = pl.pallas_call(
        _logits_body,
        grid=(N // T,),
        in_specs=[
            pl.BlockSpec((T, D), lambda i: (i, 0)),
            pl.BlockSpec((D, 2 * E), lambda i: (0, 0)),
            pl.BlockSpec((1, 2 * E), lambda i: (0, 0)),
            pl.BlockSpec((E, T), lambda i: (0, i)),
        ],
        out_specs=pl.BlockSpec((E, T), lambda i: (0, i)),
        out_shape=jax.ShapeDtypeStruct((E, N), jnp.float32),
        compiler_params=pltpu.CompilerParams(
            dimension_semantics=("arbitrary",),
        ),
    )(x, wt, b, noiseT)

    info = plsc.get_sparse_core_info()
    nw = info.num_cores * info.num_subcores
    tpw = N // nw

    route = functools.partial(
        pl.kernel,
        out_type=[
            jax.ShapeDtypeStruct((N * E,), jnp.float32),
            jax.ShapeDtypeStruct((_K, N), jnp.int32),
        ],
        scratch_types=[
            pltpu.VMEM((E * tpw,), jnp.float32),
            pltpu.VMEM((tpw * E,), jnp.float32),
            pltpu.VMEM((_K * tpw,), jnp.int32),
            pltpu.SemaphoreType.DMA,
        ],
        mesh=plsc.VectorSubcoreMesh(core_axis_name="c", subcore_axis_name="s"),
        compiler_params=pltpu.CompilerParams(needs_layout_passes=False),
    )(functools.partial(_route_body, tpw))

    outp, idxT = route(noisyT)
    return outp.reshape(B, L, E), idxT.T.reshape(B, L, _K)
